# baseline (speedup 1.0000x reference)
import jax
import jax.numpy as jnp
from jax import lax
from jax.experimental import pallas as pl
from jax.experimental.pallas import tpu as pltpu
from jax.experimental.pallas import tpu_sc as plsc

N_NODES = 10000
N_EDGES = 320000
D_FEAT = 128

NUM_CORES = 2
NUM_SUBCORES = 16
NW = NUM_CORES * NUM_SUBCORES
E_PER_W = N_EDGES // NW
CHUNK = 80
N_CHUNKS = E_PER_W // CHUNK
LANES = 16
NBUF = 3


def _dot_kernel(h_hbm, src_hbm, dst_hbm, out_hbm, *scr):
    src_v, dst_v = scr[0], scr[1]
    hs = scr[2:2 + NBUF]
    hd = scr[2 + NBUF:2 + 2 * NBUF]
    out_v = scr[2 + 2 * NBUF]
    sems = scr[3 + 2 * NBUF:]
    sems = tuple((sems[2 * b], sems[2 * b + 1]) for b in range(NBUF))

    cid = lax.axis_index("c")
    sid = lax.axis_index("s")
    wid = sid * NUM_CORES + cid

    pltpu.sync_copy(src_hbm.at[wid], src_v)
    pltpu.sync_copy(dst_hbm.at[wid], dst_v)

    def issue(i, b):
        pltpu.async_copy(h_hbm.at[src_v.at[i]], hs[b], sems[b][0])
        pltpu.async_copy(h_hbm.at[dst_v.at[i]], hd[b], sems[b][1])

    def wait(b):
        pltpu.make_async_copy(h_hbm.at[src_v.at[0]], hs[b], sems[b][0]).wait()
        pltpu.make_async_copy(h_hbm.at[dst_v.at[0]], hd[b], sems[b][1]).wait()

    lane_ids = lax.iota(jnp.int32, LANES)

    def hsum(v):
        for r in (8, 4, 2, 1):
            idx = (lane_ids + r) & (LANES - 1)
            v = v + v.at[idx].get(mode="promise_in_bounds")
        return v

    def compute(i, b):
        hs_v = hs[b]
        hd_v = hd[b]

        def group_body(g, carry2):
            scores = jnp.zeros((LANES,), jnp.float32)
            for e2 in range(LANES):
                e = g * LANES + e2
                acc = jnp.zeros((LANES,), jnp.float32)
                for k in range(D_FEAT // (2 * LANES)):
                    ai = hs_v[e, pl.ds(k * LANES, LANES)]
                    bi = hd_v[e, pl.ds(k * LANES, LANES)]
                    a_lo = plsc.bitcast(ai << 16, jnp.float32)
                    b_lo = plsc.bitcast(bi << 16, jnp.float32)
                    a_hi = plsc.bitcast(ai, jnp.float32)
                    b_hi = plsc.bitcast(bi, jnp.float32)
                    acc = acc + a_lo * b_lo
                    acc = acc + a_hi * b_hi
                scores = jnp.where(lane_ids == e2, hsum(acc), scores)
            out_v[pl.ds(i * CHUNK + g * LANES, LANES)] = scores
            return carry2

        lax.fori_loop(0, CHUNK // LANES, group_body, 0)

    for b in range(NBUF):
        issue(b, b)

    def ring_body(j, carry):
        for b in range(NBUF):
            i = NBUF * j + b
            wait(b)
            compute(i, b)
            nxt = i + NBUF

            @pl.when(nxt < N_CHUNKS)
            def _():
                issue(nxt, b)

        return carry

    lax.fori_loop(0, N_CHUNKS // NBUF, ring_body, 0)
    for t in range(N_CHUNKS % NBUF):
        wait(t)
        compute((N_CHUNKS // NBUF) * NBUF + t, t)

    pltpu.sync_copy(out_v, out_hbm.at[pl.ds(wid * E_PER_W, E_PER_W)])


@jax.jit
def _scored(h, src_rows, dst_rows):
    mesh = plsc.VectorSubcoreMesh(core_axis_name="c", subcore_axis_name="s")
    f = pl.kernel(
        _dot_kernel,
        out_type=jax.ShapeDtypeStruct((N_EDGES,), jnp.float32),
        mesh=mesh,
        compiler_params=pltpu.CompilerParams(
            needs_layout_passes=False, use_tc_tiling_on_sc=False),
        scratch_types=[
            pltpu.VMEM((N_CHUNKS, CHUNK), jnp.int32),
            pltpu.VMEM((N_CHUNKS, CHUNK), jnp.int32),
            *([pltpu.VMEM((CHUNK, D_FEAT // 2), jnp.int32)] * (2 * NBUF)),
            pltpu.VMEM((E_PER_W,), jnp.float32),
            *([pltpu.SemaphoreType.DMA] * (2 * NBUF)),
        ],
    )
    return f(h, src_rows, dst_rows)


def kernel(h, edge_index):
    hb = h.astype(jnp.bfloat16).reshape(N_NODES, D_FEAT // 2, 2)
    h32 = jax.lax.bitcast_convert_type(hb, jnp.int32)
    src = edge_index[0].astype(jnp.int32).reshape(NW, N_CHUNKS, CHUNK)
    dst = edge_index[1].astype(jnp.int32).reshape(NW, N_CHUNKS, CHUNK)
    return _scored(h32, src, dst)


# speedup vs baseline: 1.1876x; 1.1876x over previous
import jax
import jax.numpy as jnp
from jax import lax
from jax.experimental import pallas as pl
from jax.experimental.pallas import tpu as pltpu
from jax.experimental.pallas import tpu_sc as plsc

N_NODES = 10000
N_EDGES = 320000
D_FEAT = 128

NUM_CORES = 2
NUM_SUBCORES = 16
NW = NUM_CORES * NUM_SUBCORES
E_PER_W = N_EDGES // NW
CHUNK = 80
N_CHUNKS = E_PER_W // CHUNK
LANES = 16
NBUF = 2


def _dot_kernel(h_hbm, src_hbm, dst_hbm, out_hbm, *scr):
    src_v, dst_v = scr[0], scr[1]
    hs = scr[2:2 + NBUF]
    hd = scr[2 + NBUF:2 + 2 * NBUF]
    out_v = scr[2 + 2 * NBUF]
    sems = scr[3 + 2 * NBUF:]
    sems = tuple((sems[2 * b], sems[2 * b + 1]) for b in range(NBUF))

    cid = lax.axis_index("c")
    sid = lax.axis_index("s")
    wid = sid * NUM_CORES + cid

    pltpu.sync_copy(src_hbm.at[wid], src_v)
    pltpu.sync_copy(dst_hbm.at[wid], dst_v)

    def issue(i, b):
        pltpu.async_copy(h_hbm.at[src_v.at[i]], hs[b], sems[b][0])
        pltpu.async_copy(h_hbm.at[dst_v.at[i]], hd[b], sems[b][1])

    def wait(b):
        pltpu.make_async_copy(h_hbm.at[src_v.at[0]], hs[b], sems[b][0]).wait()
        pltpu.make_async_copy(h_hbm.at[dst_v.at[0]], hd[b], sems[b][1]).wait()

    lane_ids = lax.iota(jnp.int32, LANES)

    def hsum(v):
        for r in (8, 4, 2, 1):
            idx = (lane_ids + r) & (LANES - 1)
            v = v + v.at[idx].get(mode="promise_in_bounds")
        return v

    def compute(i, b):
        hs_v = hs[b]
        hd_v = hd[b]

        def group_body(g, carry2):
            scores = jnp.zeros((LANES,), jnp.float32)
            for e2 in range(LANES):
                e = g * LANES + e2
                acc = jnp.zeros((LANES,), jnp.float32)
                for k in range(D_FEAT // (2 * LANES)):
                    ai = hs_v[e, pl.ds(k * LANES, LANES)]
                    bi = hd_v[e, pl.ds(k * LANES, LANES)]
                    a_lo = plsc.bitcast(ai << 16, jnp.float32)
                    b_lo = plsc.bitcast(bi << 16, jnp.float32)
                    a_hi = plsc.bitcast(ai, jnp.float32)
                    b_hi = plsc.bitcast(bi, jnp.float32)
                    acc = acc + a_lo * b_lo
                    acc = acc + a_hi * b_hi
                scores = jnp.where(lane_ids == e2, hsum(acc), scores)
            out_v[pl.ds(i * CHUNK + g * LANES, LANES)] = scores
            return carry2

        lax.fori_loop(0, CHUNK // LANES, group_body, 0)

    for b in range(NBUF):
        issue(b, b)

    def ring_body(j, carry):
        for b in range(NBUF):
            i = NBUF * j + b
            wait(b)
            compute(i, b)
            nxt = i + NBUF

            @pl.when(nxt < N_CHUNKS)
            def _():
                issue(nxt, b)

        return carry

    lax.fori_loop(0, N_CHUNKS // NBUF, ring_body, 0)
    for t in range(N_CHUNKS % NBUF):
        wait(t)
        compute((N_CHUNKS // NBUF) * NBUF + t, t)

    pltpu.sync_copy(out_v, out_hbm.at[pl.ds(wid * E_PER_W, E_PER_W)])


@jax.jit
def _scored(h, src_rows, dst_rows):
    mesh = plsc.VectorSubcoreMesh(core_axis_name="c", subcore_axis_name="s")
    f = pl.kernel(
        _dot_kernel,
        out_type=jax.ShapeDtypeStruct((N_EDGES,), jnp.float32),
        mesh=mesh,
        compiler_params=pltpu.CompilerParams(
            needs_layout_passes=False, use_tc_tiling_on_sc=False),
        scratch_types=[
            pltpu.VMEM((N_CHUNKS, CHUNK), jnp.int32),
            pltpu.VMEM((N_CHUNKS, CHUNK), jnp.int32),
            *([pltpu.VMEM((CHUNK, D_FEAT // 2), jnp.int32)] * (2 * NBUF)),
            pltpu.VMEM((E_PER_W,), jnp.float32),
            *([pltpu.SemaphoreType.DMA] * (2 * NBUF)),
        ],
    )
    return f(h, src_rows, dst_rows)


def kernel(h, edge_index):
    hb = h.astype(jnp.bfloat16).reshape(N_NODES, D_FEAT // 2, 2)
    h32 = jax.lax.bitcast_convert_type(hb, jnp.int32)
    src = edge_index[0].astype(jnp.int32).reshape(NW, N_CHUNKS, CHUNK)
    dst = edge_index[1].astype(jnp.int32).reshape(NW, N_CHUNKS, CHUNK)
    return _scored(h32, src, dst)


# speedup vs baseline: 1.3724x; 1.1556x over previous
import jax
import jax.numpy as jnp
from jax import lax
from jax.experimental import pallas as pl
from jax.experimental.pallas import tpu as pltpu
from jax.experimental.pallas import tpu_sc as plsc

N_NODES = 10000
N_EDGES = 320000
D_FEAT = 128

NUM_CORES = 2
NUM_SUBCORES = 16
NW = NUM_CORES * NUM_SUBCORES
E_PER_W = N_EDGES // NW
CHUNK = 80
N_CHUNKS = E_PER_W // CHUNK
LANES = 16
NBUF = 2


def _dot_kernel(h_hbm, src_hbm, dst_hbm, out_hbm, *scr):
    src_v, dst_v = scr[0], scr[1]
    hs = scr[2:2 + NBUF]
    hd = scr[2 + NBUF:2 + 2 * NBUF]
    out_v = scr[2 + 2 * NBUF]
    h_sh = scr[3 + 2 * NBUF]
    sems = scr[4 + 2 * NBUF:]
    sems = tuple((sems[2 * b], sems[2 * b + 1]) for b in range(NBUF))

    cid = lax.axis_index("c")
    sid = lax.axis_index("s")
    wid = sid * NUM_CORES + cid

    rows = N_NODES // NUM_SUBCORES
    pltpu.sync_copy(h_hbm.at[pl.ds(sid * rows, rows)],
                    h_sh.at[pl.ds(sid * rows, rows)])
    pltpu.sync_copy(src_hbm.at[wid], src_v)
    pltpu.sync_copy(dst_hbm.at[wid], dst_v)
    plsc.subcore_barrier()

    def issue(i, b):
        pltpu.async_copy(h_sh.at[src_v.at[i]], hs[b], sems[b][0])
        pltpu.async_copy(h_sh.at[dst_v.at[i]], hd[b], sems[b][1])

    def wait(b):
        pltpu.make_async_copy(h_sh.at[src_v.at[0]], hs[b], sems[b][0]).wait()
        pltpu.make_async_copy(h_sh.at[dst_v.at[0]], hd[b], sems[b][1]).wait()

    lane_ids = lax.iota(jnp.int32, LANES)

    def hsum(v):
        for r in (8, 4, 2, 1):
            idx = (lane_ids + r) & (LANES - 1)
            v = v + v.at[idx].get(mode="promise_in_bounds")
        return v

    def compute(i, b):
        hs_v = hs[b]
        hd_v = hd[b]

        def group_body(g, carry2):
            scores = jnp.zeros((LANES,), jnp.float32)
            for e2 in range(LANES):
                e = g * LANES + e2
                acc = jnp.zeros((LANES,), jnp.float32)
                for k in range(D_FEAT // (2 * LANES)):
                    ai = hs_v[e, pl.ds(k * LANES, LANES)]
                    bi = hd_v[e, pl.ds(k * LANES, LANES)]
                    a_lo = plsc.bitcast(ai << 16, jnp.float32)
                    b_lo = plsc.bitcast(bi << 16, jnp.float32)
                    a_hi = plsc.bitcast(ai, jnp.float32)
                    b_hi = plsc.bitcast(bi, jnp.float32)
                    acc = acc + a_lo * b_lo
                    acc = acc + a_hi * b_hi
                scores = jnp.where(lane_ids == e2, hsum(acc), scores)
            out_v[pl.ds(i * CHUNK + g * LANES, LANES)] = scores
            return carry2

        lax.fori_loop(0, CHUNK // LANES, group_body, 0)

    for b in range(NBUF):
        issue(b, b)

    def ring_body(j, carry):
        for b in range(NBUF):
            i = NBUF * j + b
            wait(b)
            compute(i, b)
            nxt = i + NBUF

            @pl.when(nxt < N_CHUNKS)
            def _():
                issue(nxt, b)

        return carry

    lax.fori_loop(0, N_CHUNKS // NBUF, ring_body, 0)
    for t in range(N_CHUNKS % NBUF):
        wait(t)
        compute((N_CHUNKS // NBUF) * NBUF + t, t)

    pltpu.sync_copy(out_v, out_hbm.at[pl.ds(wid * E_PER_W, E_PER_W)])


@jax.jit
def _scored(h, src_rows, dst_rows):
    mesh = plsc.VectorSubcoreMesh(core_axis_name="c", subcore_axis_name="s")
    f = pl.kernel(
        _dot_kernel,
        out_type=jax.ShapeDtypeStruct((N_EDGES,), jnp.float32),
        mesh=mesh,
        compiler_params=pltpu.CompilerParams(
            needs_layout_passes=False, use_tc_tiling_on_sc=False),
        scratch_types=[
            pltpu.VMEM((N_CHUNKS, CHUNK), jnp.int32),
            pltpu.VMEM((N_CHUNKS, CHUNK), jnp.int32),
            *([pltpu.VMEM((CHUNK, D_FEAT // 2), jnp.int32)] * (2 * NBUF)),
            pltpu.VMEM((E_PER_W,), jnp.float32),
            pltpu.VMEM_SHARED((N_NODES, D_FEAT // 2), jnp.int32),
            *([pltpu.SemaphoreType.DMA] * (2 * NBUF)),
        ],
    )
    return f(h, src_rows, dst_rows)


def kernel(h, edge_index):
    hb = h.astype(jnp.bfloat16).reshape(N_NODES, D_FEAT // 2, 2)
    h32 = jax.lax.bitcast_convert_type(hb, jnp.int32)
    src = edge_index[0].astype(jnp.int32).reshape(NW, N_CHUNKS, CHUNK)
    dst = edge_index[1].astype(jnp.int32).reshape(NW, N_CHUNKS, CHUNK)
    return _scored(h32, src, dst)
